# baseline (speedup 1.0000x reference)
import functools

import jax
import jax.numpy as jnp
from jax import lax
from jax.experimental import pallas as pl
from jax.experimental.pallas import tpu as pltpu
from jax.experimental.pallas import tpu_sc as plsc

_NUM_SC_CORES = 2
_NUM_SC_SUBCORES = 16
_NCH = 5
_G_NBUF = 2
_S_NBUF = 2
_SCATTER_W = 40
_EDGE_BLK = 2000


def _proj_body(x_ref, w_ref, out_ref):
    out_ref[0] = jnp.dot(x_ref[...], w_ref[0],
                         preferred_element_type=jnp.float32)


def _proj(x_pad, wstack, block=2048):
    n_pad, c = x_pad.shape
    return pl.pallas_call(
        _proj_body,
        grid=(2, n_pad // block),
        in_specs=[
            pl.BlockSpec((block, c), lambda i, j: (j, 0)),
            pl.BlockSpec((1, c, c), lambda i, j: (i, 0, 0)),
        ],
        out_specs=pl.BlockSpec((1, block, c), lambda i, j: (i, j, 0)),
        out_shape=jax.ShapeDtypeStruct((2, n_pad, c), jnp.float32),
    )(x_pad, wstack)


def _sc_gather(tables, idxp, kc, e_ch):
    _, n_pad, c = tables.shape
    e_pad = idxp.shape[1]
    w = 128
    ns = _NUM_SC_SUBCORES
    nb = _G_NBUF
    per_sub = e_pad // (ns * w)
    iters = per_sub // nb
    rows_tab = n_pad // ns
    mesh = plsc.VectorSubcoreMesh(core_axis_name="c", subcore_axis_name="s")

    scratch = ([pltpu.VMEM((w,), jnp.int32)] * nb
               + [pltpu.VMEM((w, c), jnp.float32)] * nb
               + [pltpu.SemaphoreType.DMA] * (2 * nb)
               + [pltpu.VMEM_SHARED((n_pad, c), jnp.float32)])

    @functools.partial(
        pl.kernel,
        out_type=jax.ShapeDtypeStruct((_NUM_SC_CORES, e_ch, c), jnp.float32),
        mesh=mesh,
        scratch_types=scratch,
    )
    def k(tab_hbm, idx_hbm, out_hbm, *sc):
        idx_v = sc[0:nb]
        rows_v = sc[nb:2 * nb]
        isem = sc[2 * nb:3 * nb]
        osem = sc[3 * nb:4 * nb]
        tab_sh = sc[4 * nb]
        cid = lax.axis_index("c")
        sid = lax.axis_index("s")
        pltpu.sync_copy(tab_hbm.at[cid].at[pl.ds(sid * rows_tab, rows_tab)],
                        tab_sh.at[pl.ds(sid * rows_tab, rows_tab)])
        plsc.subcore_barrier()
        start = sid * per_sub

        for u in range(nb):
            pltpu.async_copy(
                idx_hbm.at[cid * _NCH + kc].at[pl.ds((start + u) * w, w)], idx_v[u],
                isem[u])

        @pl.loop(0, iters)
        def _(ci):
            for u in range(nb):
                base = (start + ci * nb + u) * w

                @pl.when(jnp.logical_and(ci > 0, base - nb * w < e_ch))
                def _():
                    pltpu.make_async_copy(
                        rows_v[u],
                        out_hbm.at[cid].at[pl.ds(base - nb * w, w)],
                        osem[u]).wait()

                pltpu.make_async_copy(
                    idx_hbm.at[cid * _NCH + kc].at[pl.ds(base, w)], idx_v[u],
                    isem[u]).wait()
                pltpu.sync_copy(tab_sh.at[idx_v[u]], rows_v[u])

                @pl.when(base < e_ch)
                def _():
                    pltpu.async_copy(
                        rows_v[u], out_hbm.at[cid].at[pl.ds(base, w)],
                        osem[u])

                @pl.when(ci < iters - 1)
                def _():
                    pltpu.async_copy(
                        idx_hbm.at[cid * _NCH + kc].at[pl.ds(base + nb * w, w)],
                        idx_v[u], isem[u])

        for u in range(nb):
            last = (start + (iters - 1) * nb + u) * w

            @pl.when(last < e_ch)
            def _():
                pltpu.make_async_copy(
                    rows_v[u], out_hbm.at[cid].at[pl.ds(last, w)],
                    osem[u]).wait()

    return k(tables, idxp)


def _edge_body(ga_ref, gb_ref, ea_ref, w1_ref, b1_ref, w2_ref, b2_ref,
               g_ref, bt_ref, out_ref):
    ea = ea_ref[...]
    pre = (ga_ref[0] + gb_ref[0]
           + jnp.dot(ea, w1_ref[...], preferred_element_type=jnp.float32)
           + b1_ref[...])
    h = pre * jax.nn.sigmoid(pre)
    h2 = jnp.dot(h, w2_ref[...], preferred_element_type=jnp.float32) + b2_ref[...]
    mu = jnp.mean(h2, axis=-1, keepdims=True)
    zc = h2 - mu
    var = jnp.mean(zc * zc, axis=-1, keepdims=True)
    out_ref[...] = zc * lax.rsqrt(var + 1e-5) * g_ref[...] + bt_ref[...] + ea


def _edge_mlp(gab, ea_arr, ea_off_blocks, w1c, b1, w2, b2, g, bt):
    block = _EDGE_BLK
    _, e_ch, c = gab.shape
    row = lambda v: v.reshape(1, c)
    aspec = pl.BlockSpec((1, block, c), lambda i: (0, i, 0))
    bspec2 = pl.BlockSpec((1, block, c), lambda i: (1, i, 0))
    easpec = pl.BlockSpec((block, c), lambda i: (i + ea_off_blocks, 0))
    ospec = pl.BlockSpec((block, c), lambda i: (i, 0))
    wspec = pl.BlockSpec((c, c), lambda i: (0, 0))
    vspec = pl.BlockSpec((1, c), lambda i: (0, 0))
    return pl.pallas_call(
        _edge_body,
        grid=(e_ch // block,),
        in_specs=[aspec, bspec2, easpec, wspec, vspec, wspec, vspec, vspec,
                  vspec],
        out_specs=ospec,
        out_shape=jax.ShapeDtypeStruct((e_ch, c), jnp.float32),
    )(gab, gab, ea_arr, w1c, row(b1), w2, row(b2), row(g), row(bt))


def _sc_scatter(edges, dst, kc, zeros):
    e_ch, c = edges.shape
    n = zeros.shape[0]
    w = _SCATTER_W
    nb = _S_NBUF
    nw = _NUM_SC_CORES * _NUM_SC_SUBCORES
    per_tile = e_ch // nw
    chunks = per_tile // w
    iters = chunks // nb
    assert chunks % nb == 0 and per_tile % w == 0
    rows_per_sub = n // _NUM_SC_SUBCORES
    assert rows_per_sub % 8 == 0
    mesh = plsc.VectorSubcoreMesh(core_axis_name="c", subcore_axis_name="s")

    scratch = ([pltpu.VMEM((w,), jnp.int32)] * nb
               + [pltpu.VMEM((w, c), jnp.float32)] * nb
               + [pltpu.SemaphoreType.DMA] * (2 * nb)
               + [pltpu.VMEM_SHARED((n, c), jnp.float32)])

    @functools.partial(
        pl.kernel,
        out_type=jax.ShapeDtypeStruct((_NUM_SC_CORES, n, c), jnp.float32),
        mesh=mesh,
        scratch_types=scratch,
    )
    def k(edges_hbm, dst_hbm, zeros_hbm, out_hbm, *sc):
        idx_v = sc[0:nb]
        rows_v = sc[nb:2 * nb]
        isem = sc[2 * nb:3 * nb]
        esem = sc[3 * nb:4 * nb]
        acc_sh = sc[4 * nb]
        cid = lax.axis_index("c")
        sid = lax.axis_index("s")
        wid = sid * _NUM_SC_CORES + cid
        row0 = sid * rows_per_sub
        pltpu.sync_copy(zeros_hbm.at[pl.ds(row0, rows_per_sub)],
                        acc_sh.at[pl.ds(row0, rows_per_sub)])
        plsc.subcore_barrier()
        tbase = wid * per_tile
        dbase = kc * e_ch + tbase

        for u in range(nb):
            pltpu.async_copy(dst_hbm.at[pl.ds(dbase + u * w, w)], idx_v[u],
                             isem[u])
            pltpu.async_copy(edges_hbm.at[pl.ds(tbase + u * w, w)], rows_v[u],
                             esem[u])

        @pl.loop(0, iters)
        def _(ci):
            for u in range(nb):
                off = (ci * nb + u) * w
                pltpu.make_async_copy(dst_hbm.at[pl.ds(dbase + off, w)],
                                      idx_v[u], isem[u]).wait()
                pltpu.make_async_copy(edges_hbm.at[pl.ds(tbase + off, w)],
                                      rows_v[u], esem[u]).wait()
                pltpu.sync_copy(rows_v[u], acc_sh.at[idx_v[u]], add=True)

                @pl.when(ci < iters - 1)
                def _():
                    pltpu.async_copy(
                        dst_hbm.at[pl.ds(dbase + off + nb * w, w)], idx_v[u],
                        isem[u])
                    pltpu.async_copy(
                        edges_hbm.at[pl.ds(tbase + off + nb * w, w)],
                        rows_v[u], esem[u])

        plsc.subcore_barrier()
        pltpu.sync_copy(acc_sh.at[pl.ds(row0, rows_per_sub)],
                        out_hbm.at[cid].at[pl.ds(row0, rows_per_sub)])

    return k(edges, dst, zeros)


def _node_body(*refs):
    x_ref = refs[0]
    agg_refs = refs[1:1 + 2 * _NCH]
    (wa_ref, wb_ref, b1_ref, w2_ref, b2_ref, g_ref, bt_ref,
     out_ref) = refs[1 + 2 * _NCH:]
    x = x_ref[...]
    agg = agg_refs[0][0]
    for r in agg_refs[1:]:
        agg = agg + r[0]
    pre = (jnp.dot(x, wa_ref[...], preferred_element_type=jnp.float32)
           + jnp.dot(agg, wb_ref[...], preferred_element_type=jnp.float32)
           + b1_ref[...])
    h = pre * jax.nn.sigmoid(pre)
    h2 = jnp.dot(h, w2_ref[...], preferred_element_type=jnp.float32) + b2_ref[...]
    mu = jnp.mean(h2, axis=-1, keepdims=True)
    zc = h2 - mu
    var = jnp.mean(zc * zc, axis=-1, keepdims=True)
    out_ref[...] = zc * lax.rsqrt(var + 1e-5) * g_ref[...] + bt_ref[...] + x


def _node_mlp(x, aggs, wa, wb, b1, w2, b2, g, bt, block=2000):
    n, c = x.shape
    row = lambda v: v.reshape(1, c)
    bspec = pl.BlockSpec((block, c), lambda i: (i, 0))
    a0 = pl.BlockSpec((1, block, c), lambda i: (0, i, 0))
    a1 = pl.BlockSpec((1, block, c), lambda i: (1, i, 0))
    wspec = pl.BlockSpec((c, c), lambda i: (0, 0))
    vspec = pl.BlockSpec((1, c), lambda i: (0, 0))
    agg_args = []
    agg_specs = []
    for a in aggs:
        agg_args += [a, a]
        agg_specs += [a0, a1]
    return pl.pallas_call(
        _node_body,
        grid=(n // block,),
        in_specs=[bspec] + agg_specs + [wspec, wspec, vspec, wspec, vspec,
                                        vspec, vspec],
        out_specs=bspec,
        out_shape=jax.ShapeDtypeStruct((n, c), jnp.float32),
    )(x, *agg_args, wa, wb, row(b1), w2, row(b2), row(g), row(bt))


def kernel(x, edge_attr, edge_index, shapes, e_W1, e_b1, e_W2, e_b2, e_g,
           e_bt, n_W1, n_b1, n_W2, n_b2, n_g, n_bt):
    n, c = x.shape
    num_layers = e_W1.shape[0]
    src = edge_index[0]
    dst = edge_index[1]
    e = src.shape[0]
    e_ch = e // _NCH
    assert e % (_NCH * 128) == 0 and e_ch % _EDGE_BLK == 0
    n_pad = ((n + 2047) // 2048) * 2048
    stride = 128 * _NUM_SC_SUBCORES * _G_NBUF
    e_pad = ((e_ch + stride - 1) // stride) * stride
    idxp = jnp.pad(jnp.stack([dst, src]).reshape(2 * _NCH, e_ch),
                   ((0, 0), (0, e_pad - e_ch)))
    zeros = jnp.zeros((n_pad, c), jnp.float32)
    x_out = x
    ea_chunks = None
    for l in range(num_layers):
        w1 = e_W1[l]
        x_pad = jnp.pad(x_out, ((0, n_pad - n), (0, 0)))
        tables = _proj(x_pad, jnp.stack([w1[:c], w1[c:2 * c]]))
        new_chunks = []
        aggs = []
        for kc in range(_NCH):
            gab = _sc_gather(tables, idxp, kc, e_ch)
            if ea_chunks is None:
                ea_arr, off = edge_attr, kc * (e_ch // _EDGE_BLK)
            else:
                ea_arr, off = ea_chunks[kc], 0
            ea_new = _edge_mlp(gab, ea_arr, off, w1[2 * c:], e_b1[l],
                               e_W2[l], e_b2[l], e_g[l], e_bt[l])
            new_chunks.append(ea_new)
            aggs.append(_sc_scatter(ea_new, dst, kc, zeros))
        nw1 = n_W1[l]
        x_out = _node_mlp(x_out, aggs, nw1[:c], nw1[c:], n_b1[l], n_W2[l],
                          n_b2[l], n_g[l], n_bt[l])
        ea_chunks = new_chunks
    return (x_out, jnp.concatenate(ea_chunks, axis=0))


# speedup vs baseline: 1.0670x; 1.0670x over previous
import functools

import jax
import jax.numpy as jnp
from jax import lax
from jax.experimental import pallas as pl
from jax.experimental.pallas import tpu as pltpu
from jax.experimental.pallas import tpu_sc as plsc

_NUM_SC_CORES = 2
_NUM_SC_SUBCORES = 16
_NCH = 5
_G_NBUF = 2
_S_NBUF = 2
_SCATTER_W = 40
_EDGE_BLK = 2000


def _proj_body(x_ref, w_ref, out_ref):
    out_ref[0] = jnp.dot(x_ref[...], w_ref[0],
                         preferred_element_type=jnp.float32)


def _proj(x_pad, wstack, block=2048):
    n_pad, c = x_pad.shape
    return pl.pallas_call(
        _proj_body,
        grid=(2, n_pad // block),
        in_specs=[
            pl.BlockSpec((block, c), lambda i, j: (j, 0)),
            pl.BlockSpec((1, c, c), lambda i, j: (i, 0, 0)),
        ],
        out_specs=pl.BlockSpec((1, block, c), lambda i, j: (i, j, 0)),
        out_shape=jax.ShapeDtypeStruct((2, n_pad, c), jnp.float32),
    )(x_pad, wstack)


def _sc_gather(tables, idxp, kc, e_ch):
    _, n_pad, c = tables.shape
    e_pad = idxp.shape[1]
    w = 128
    ns = _NUM_SC_SUBCORES
    nb = _G_NBUF
    per_sub = e_pad // (ns * w)
    iters = per_sub // nb
    rows_tab = n_pad // ns
    mesh = plsc.VectorSubcoreMesh(core_axis_name="c", subcore_axis_name="s")

    scratch = ([pltpu.VMEM((w,), jnp.int32)] * nb
               + [pltpu.VMEM((w, c), jnp.float32)] * nb
               + [pltpu.SemaphoreType.DMA] * (2 * nb)
               + [pltpu.VMEM_SHARED((n_pad, c), jnp.float32)])

    @functools.partial(
        pl.kernel,
        out_type=jax.ShapeDtypeStruct((_NUM_SC_CORES, e_ch, c), jnp.float32),
        mesh=mesh,
        scratch_types=scratch,
    )
    def k(tab_hbm, idx_hbm, out_hbm, *sc):
        idx_v = sc[0:nb]
        rows_v = sc[nb:2 * nb]
        isem = sc[2 * nb:3 * nb]
        osem = sc[3 * nb:4 * nb]
        tab_sh = sc[4 * nb]
        cid = lax.axis_index("c")
        sid = lax.axis_index("s")
        pltpu.sync_copy(tab_hbm.at[cid].at[pl.ds(sid * rows_tab, rows_tab)],
                        tab_sh.at[pl.ds(sid * rows_tab, rows_tab)])
        plsc.subcore_barrier()
        start = sid * per_sub

        for u in range(nb):
            pltpu.async_copy(
                idx_hbm.at[cid * _NCH + kc].at[pl.ds((start + u) * w, w)], idx_v[u],
                isem[u])

        @pl.loop(0, iters)
        def _(ci):
            for u in range(nb):
                base = (start + ci * nb + u) * w

                @pl.when(jnp.logical_and(ci > 0, base - nb * w < e_ch))
                def _():
                    pltpu.make_async_copy(
                        rows_v[u],
                        out_hbm.at[cid].at[pl.ds(base - nb * w, w)],
                        osem[u]).wait()

                pltpu.make_async_copy(
                    idx_hbm.at[cid * _NCH + kc].at[pl.ds(base, w)], idx_v[u],
                    isem[u]).wait()
                pltpu.sync_copy(tab_sh.at[idx_v[u]], rows_v[u])

                @pl.when(base < e_ch)
                def _():
                    pltpu.async_copy(
                        rows_v[u], out_hbm.at[cid].at[pl.ds(base, w)],
                        osem[u])

                @pl.when(ci < iters - 1)
                def _():
                    pltpu.async_copy(
                        idx_hbm.at[cid * _NCH + kc].at[pl.ds(base + nb * w, w)],
                        idx_v[u], isem[u])

        for u in range(nb):
            last = (start + (iters - 1) * nb + u) * w

            @pl.when(last < e_ch)
            def _():
                pltpu.make_async_copy(
                    rows_v[u], out_hbm.at[cid].at[pl.ds(last, w)],
                    osem[u]).wait()

    return k(tables, idxp)


def _alloc_body(out_ref):
    out_ref[...] = jnp.zeros_like(out_ref)


def _alloc_edges(e, c):
    return pl.pallas_call(
        _alloc_body,
        grid=(1,),
        out_specs=pl.BlockSpec((8, c), lambda i: (0, 0)),
        out_shape=jax.ShapeDtypeStruct((e, c), jnp.float32),
    )()


def _edge_body(seed_ref, ga_ref, gb_ref, ea_ref, w1_ref, b1_ref, w2_ref,
               b2_ref, g_ref, bt_ref, out_ref):
    del seed_ref
    ea = ea_ref[...]
    pre = (ga_ref[0] + gb_ref[0]
           + jnp.dot(ea, w1_ref[...], preferred_element_type=jnp.float32)
           + b1_ref[...])
    h = pre * jax.nn.sigmoid(pre)
    h2 = jnp.dot(h, w2_ref[...], preferred_element_type=jnp.float32) + b2_ref[...]
    mu = jnp.mean(h2, axis=-1, keepdims=True)
    zc = h2 - mu
    var = jnp.mean(zc * zc, axis=-1, keepdims=True)
    out_ref[...] = zc * lax.rsqrt(var + 1e-5) * g_ref[...] + bt_ref[...] + ea


def _edge_mlp(out_buf, gab, ea_arr, off_blocks, w1c, b1, w2, b2, g, bt):
    block = _EDGE_BLK
    _, e_ch, c = gab.shape
    e = ea_arr.shape[0]
    row = lambda v: v.reshape(1, c)
    seedspec = pl.BlockSpec((8, c), lambda i: (0, 0))
    aspec = pl.BlockSpec((1, block, c), lambda i: (0, i, 0))
    bspec2 = pl.BlockSpec((1, block, c), lambda i: (1, i, 0))
    easpec = pl.BlockSpec((block, c), lambda i: (i + off_blocks, 0))
    ospec = pl.BlockSpec((block, c), lambda i: (i + off_blocks, 0))
    wspec = pl.BlockSpec((c, c), lambda i: (0, 0))
    vspec = pl.BlockSpec((1, c), lambda i: (0, 0))
    return pl.pallas_call(
        _edge_body,
        grid=(e_ch // block,),
        in_specs=[seedspec, aspec, bspec2, easpec, wspec, vspec, wspec,
                  vspec, vspec, vspec],
        out_specs=ospec,
        out_shape=jax.ShapeDtypeStruct((e, c), jnp.float32),
        input_output_aliases={0: 0},
    )(out_buf, gab, gab, ea_arr, w1c, row(b1), w2, row(b2), row(g), row(bt))


def _sc_scatter(edges, dst, zeros):
    e, c = edges.shape
    n = zeros.shape[0]
    w = _SCATTER_W
    nb = _S_NBUF
    nw = _NUM_SC_CORES * _NUM_SC_SUBCORES
    per_tile = e // nw
    chunks = per_tile // w
    iters = chunks // nb
    assert chunks % nb == 0 and per_tile % w == 0
    rows_per_sub = n // _NUM_SC_SUBCORES
    assert rows_per_sub % 8 == 0
    mesh = plsc.VectorSubcoreMesh(core_axis_name="c", subcore_axis_name="s")

    scratch = ([pltpu.VMEM((w,), jnp.int32)] * nb
               + [pltpu.VMEM((w, c), jnp.float32)] * nb
               + [pltpu.SemaphoreType.DMA] * (2 * nb)
               + [pltpu.VMEM_SHARED((n, c), jnp.float32)])

    @functools.partial(
        pl.kernel,
        out_type=jax.ShapeDtypeStruct((_NUM_SC_CORES, n, c), jnp.float32),
        mesh=mesh,
        scratch_types=scratch,
    )
    def k(edges_hbm, dst_hbm, zeros_hbm, out_hbm, *sc):
        idx_v = sc[0:nb]
        rows_v = sc[nb:2 * nb]
        isem = sc[2 * nb:3 * nb]
        esem = sc[3 * nb:4 * nb]
        acc_sh = sc[4 * nb]
        cid = lax.axis_index("c")
        sid = lax.axis_index("s")
        wid = sid * _NUM_SC_CORES + cid
        row0 = sid * rows_per_sub
        pltpu.sync_copy(zeros_hbm.at[pl.ds(row0, rows_per_sub)],
                        acc_sh.at[pl.ds(row0, rows_per_sub)])
        plsc.subcore_barrier()
        tbase = wid * per_tile
        dbase = tbase

        for u in range(nb):
            pltpu.async_copy(dst_hbm.at[pl.ds(dbase + u * w, w)], idx_v[u],
                             isem[u])
            pltpu.async_copy(edges_hbm.at[pl.ds(tbase + u * w, w)], rows_v[u],
                             esem[u])

        @pl.loop(0, iters)
        def _(ci):
            for u in range(nb):
                off = (ci * nb + u) * w
                pltpu.make_async_copy(dst_hbm.at[pl.ds(dbase + off, w)],
                                      idx_v[u], isem[u]).wait()
                pltpu.make_async_copy(edges_hbm.at[pl.ds(tbase + off, w)],
                                      rows_v[u], esem[u]).wait()
                pltpu.sync_copy(rows_v[u], acc_sh.at[idx_v[u]], add=True)

                @pl.when(ci < iters - 1)
                def _():
                    pltpu.async_copy(
                        dst_hbm.at[pl.ds(dbase + off + nb * w, w)], idx_v[u],
                        isem[u])
                    pltpu.async_copy(
                        edges_hbm.at[pl.ds(tbase + off + nb * w, w)],
                        rows_v[u], esem[u])

        plsc.subcore_barrier()
        pltpu.sync_copy(acc_sh.at[pl.ds(row0, rows_per_sub)],
                        out_hbm.at[cid].at[pl.ds(row0, rows_per_sub)])

    return k(edges, dst, zeros)


def _node_body(*refs):
    x_ref = refs[0]
    agg_refs = refs[1:-8]
    (wa_ref, wb_ref, b1_ref, w2_ref, b2_ref, g_ref, bt_ref,
     out_ref) = refs[-8:]
    x = x_ref[...]
    agg = agg_refs[0][0]
    for r in agg_refs[1:]:
        agg = agg + r[0]
    pre = (jnp.dot(x, wa_ref[...], preferred_element_type=jnp.float32)
           + jnp.dot(agg, wb_ref[...], preferred_element_type=jnp.float32)
           + b1_ref[...])
    h = pre * jax.nn.sigmoid(pre)
    h2 = jnp.dot(h, w2_ref[...], preferred_element_type=jnp.float32) + b2_ref[...]
    mu = jnp.mean(h2, axis=-1, keepdims=True)
    zc = h2 - mu
    var = jnp.mean(zc * zc, axis=-1, keepdims=True)
    out_ref[...] = zc * lax.rsqrt(var + 1e-5) * g_ref[...] + bt_ref[...] + x


def _node_mlp(x, aggs, wa, wb, b1, w2, b2, g, bt, block=2000):
    n, c = x.shape
    row = lambda v: v.reshape(1, c)
    bspec = pl.BlockSpec((block, c), lambda i: (i, 0))
    a0 = pl.BlockSpec((1, block, c), lambda i: (0, i, 0))
    a1 = pl.BlockSpec((1, block, c), lambda i: (1, i, 0))
    wspec = pl.BlockSpec((c, c), lambda i: (0, 0))
    vspec = pl.BlockSpec((1, c), lambda i: (0, 0))
    agg_args = []
    agg_specs = []
    for a in aggs:
        agg_args += [a, a]
        agg_specs += [a0, a1]
    return pl.pallas_call(
        _node_body,
        grid=(n // block,),
        in_specs=[bspec] + agg_specs + [wspec, wspec, vspec, wspec, vspec,
                                        vspec, vspec],
        out_specs=bspec,
        out_shape=jax.ShapeDtypeStruct((n, c), jnp.float32),
    )(x, *agg_args, wa, wb, row(b1), w2, row(b2), row(g), row(bt))


def kernel(x, edge_attr, edge_index, shapes, e_W1, e_b1, e_W2, e_b2, e_g,
           e_bt, n_W1, n_b1, n_W2, n_b2, n_g, n_bt):
    n, c = x.shape
    num_layers = e_W1.shape[0]
    src = edge_index[0]
    dst = edge_index[1]
    e = src.shape[0]
    e_ch = e // _NCH
    assert e % (_NCH * 128) == 0 and e_ch % _EDGE_BLK == 0
    n_pad = ((n + 2047) // 2048) * 2048
    stride = 128 * _NUM_SC_SUBCORES * _G_NBUF
    e_pad = ((e_ch + stride - 1) // stride) * stride
    idxp = jnp.pad(jnp.stack([dst, src]).reshape(2 * _NCH, e_ch),
                   ((0, 0), (0, e_pad - e_ch)))
    zeros = jnp.zeros((n_pad, c), jnp.float32)
    x_out = x
    ea = edge_attr
    for l in range(num_layers):
        w1 = e_W1[l]
        x_pad = jnp.pad(x_out, ((0, n_pad - n), (0, 0)))
        tables = _proj(x_pad, jnp.stack([w1[:c], w1[c:2 * c]]))
        buf = _alloc_edges(e, c)
        for kc in range(_NCH):
            gab = _sc_gather(tables, idxp, kc, e_ch)
            buf = _edge_mlp(buf, gab, ea, kc * (e_ch // _EDGE_BLK),
                            w1[2 * c:], e_b1[l], e_W2[l], e_b2[l], e_g[l],
                            e_bt[l])
        agg = _sc_scatter(buf, dst, zeros)
        nw1 = n_W1[l]
        x_out = _node_mlp(x_out, [agg], nw1[:c], nw1[c:], n_b1[l], n_W2[l],
                          n_b2[l], n_g[l], n_bt[l])
        ea = buf
    return (x_out, ea)


# speedup vs baseline: 1.0693x; 1.0021x over previous
import functools

import jax
import jax.numpy as jnp
from jax import lax
from jax.experimental import pallas as pl
from jax.experimental.pallas import tpu as pltpu
from jax.experimental.pallas import tpu_sc as plsc

_NUM_SC_CORES = 2
_NUM_SC_SUBCORES = 16
_NCH = 4
_G_NBUF = 2
_S_NBUF = 2
_SCATTER_W = 40
_EDGE_BLK = 2000


def _proj_body(x_ref, w_ref, out_ref):
    out_ref[0] = jnp.dot(x_ref[...], w_ref[0],
                         preferred_element_type=jnp.float32)


def _proj(x_pad, wstack, block=2048):
    n_pad, c = x_pad.shape
    return pl.pallas_call(
        _proj_body,
        grid=(2, n_pad // block),
        in_specs=[
            pl.BlockSpec((block, c), lambda i, j: (j, 0)),
            pl.BlockSpec((1, c, c), lambda i, j: (i, 0, 0)),
        ],
        out_specs=pl.BlockSpec((1, block, c), lambda i, j: (i, j, 0)),
        out_shape=jax.ShapeDtypeStruct((2, n_pad, c), jnp.float32),
    )(x_pad, wstack)


def _sc_gather(tables, idxp, kc, e_ch):
    _, n_pad, c = tables.shape
    e_pad = idxp.shape[1]
    w = 128
    ns = _NUM_SC_SUBCORES
    nb = _G_NBUF
    per_sub = e_pad // (ns * w)
    iters = per_sub // nb
    rows_tab = n_pad // ns
    mesh = plsc.VectorSubcoreMesh(core_axis_name="c", subcore_axis_name="s")

    scratch = ([pltpu.VMEM((w,), jnp.int32)] * nb
               + [pltpu.VMEM((w, c), jnp.float32)] * nb
               + [pltpu.SemaphoreType.DMA] * (2 * nb)
               + [pltpu.VMEM_SHARED((n_pad, c), jnp.float32)])

    @functools.partial(
        pl.kernel,
        out_type=jax.ShapeDtypeStruct((_NUM_SC_CORES, e_ch, c), jnp.float32),
        mesh=mesh,
        scratch_types=scratch,
    )
    def k(tab_hbm, idx_hbm, out_hbm, *sc):
        idx_v = sc[0:nb]
        rows_v = sc[nb:2 * nb]
        isem = sc[2 * nb:3 * nb]
        osem = sc[3 * nb:4 * nb]
        tab_sh = sc[4 * nb]
        cid = lax.axis_index("c")
        sid = lax.axis_index("s")
        pltpu.sync_copy(tab_hbm.at[cid].at[pl.ds(sid * rows_tab, rows_tab)],
                        tab_sh.at[pl.ds(sid * rows_tab, rows_tab)])
        plsc.subcore_barrier()
        start = sid * per_sub

        for u in range(nb):
            pltpu.async_copy(
                idx_hbm.at[cid * _NCH + kc].at[pl.ds((start + u) * w, w)], idx_v[u],
                isem[u])

        @pl.loop(0, iters)
        def _(ci):
            for u in range(nb):
                base = (start + ci * nb + u) * w

                @pl.when(jnp.logical_and(ci > 0, base - nb * w < e_ch))
                def _():
                    pltpu.make_async_copy(
                        rows_v[u],
                        out_hbm.at[cid].at[pl.ds(base - nb * w, w)],
                        osem[u]).wait()

                pltpu.make_async_copy(
                    idx_hbm.at[cid * _NCH + kc].at[pl.ds(base, w)], idx_v[u],
                    isem[u]).wait()
                pltpu.sync_copy(tab_sh.at[idx_v[u]], rows_v[u])

                @pl.when(base < e_ch)
                def _():
                    pltpu.async_copy(
                        rows_v[u], out_hbm.at[cid].at[pl.ds(base, w)],
                        osem[u])

                @pl.when(ci < iters - 1)
                def _():
                    pltpu.async_copy(
                        idx_hbm.at[cid * _NCH + kc].at[pl.ds(base + nb * w, w)],
                        idx_v[u], isem[u])

        for u in range(nb):
            last = (start + (iters - 1) * nb + u) * w

            @pl.when(last < e_ch)
            def _():
                pltpu.make_async_copy(
                    rows_v[u], out_hbm.at[cid].at[pl.ds(last, w)],
                    osem[u]).wait()

    return k(tables, idxp)


def _alloc_body(out_ref):
    out_ref[...] = jnp.zeros_like(out_ref)


def _alloc_edges(e, c):
    return pl.pallas_call(
        _alloc_body,
        grid=(1,),
        out_specs=pl.BlockSpec((8, c), lambda i: (0, 0)),
        out_shape=jax.ShapeDtypeStruct((e, c), jnp.float32),
    )()


def _edge_body(seed_ref, ga_ref, gb_ref, ea_ref, w1_ref, b1_ref, w2_ref,
               b2_ref, g_ref, bt_ref, out_ref):
    del seed_ref
    ea = ea_ref[...]
    pre = (ga_ref[0] + gb_ref[0]
           + jnp.dot(ea, w1_ref[...], preferred_element_type=jnp.float32)
           + b1_ref[...])
    h = pre * jax.nn.sigmoid(pre)
    h2 = jnp.dot(h, w2_ref[...], preferred_element_type=jnp.float32) + b2_ref[...]
    mu = jnp.mean(h2, axis=-1, keepdims=True)
    zc = h2 - mu
    var = jnp.mean(zc * zc, axis=-1, keepdims=True)
    out_ref[...] = zc * lax.rsqrt(var + 1e-5) * g_ref[...] + bt_ref[...] + ea


def _edge_mlp(out_buf, gab, ea_arr, off_blocks, w1c, b1, w2, b2, g, bt):
    block = _EDGE_BLK
    _, e_ch, c = gab.shape
    e = ea_arr.shape[0]
    row = lambda v: v.reshape(1, c)
    seedspec = pl.BlockSpec((8, c), lambda i: (0, 0))
    aspec = pl.BlockSpec((1, block, c), lambda i: (0, i, 0))
    bspec2 = pl.BlockSpec((1, block, c), lambda i: (1, i, 0))
    easpec = pl.BlockSpec((block, c), lambda i: (i + off_blocks, 0))
    ospec = pl.BlockSpec((block, c), lambda i: (i + off_blocks, 0))
    wspec = pl.BlockSpec((c, c), lambda i: (0, 0))
    vspec = pl.BlockSpec((1, c), lambda i: (0, 0))
    return pl.pallas_call(
        _edge_body,
        grid=(e_ch // block,),
        in_specs=[seedspec, aspec, bspec2, easpec, wspec, vspec, wspec,
                  vspec, vspec, vspec],
        out_specs=ospec,
        out_shape=jax.ShapeDtypeStruct((e, c), jnp.float32),
        input_output_aliases={0: 0},
    )(out_buf, gab, gab, ea_arr, w1c, row(b1), w2, row(b2), row(g), row(bt))


def _sc_scatter(edges, dst, zeros):
    e, c = edges.shape
    n = zeros.shape[0]
    w = _SCATTER_W
    nb = _S_NBUF
    nw = _NUM_SC_CORES * _NUM_SC_SUBCORES
    per_tile = e // nw
    chunks = per_tile // w
    iters = chunks // nb
    assert chunks % nb == 0 and per_tile % w == 0
    rows_per_sub = n // _NUM_SC_SUBCORES
    assert rows_per_sub % 8 == 0
    mesh = plsc.VectorSubcoreMesh(core_axis_name="c", subcore_axis_name="s")

    scratch = ([pltpu.VMEM((w,), jnp.int32)] * nb
               + [pltpu.VMEM((w, c), jnp.float32)] * nb
               + [pltpu.SemaphoreType.DMA] * (2 * nb)
               + [pltpu.VMEM_SHARED((n, c), jnp.float32)])

    @functools.partial(
        pl.kernel,
        out_type=jax.ShapeDtypeStruct((_NUM_SC_CORES, n, c), jnp.float32),
        mesh=mesh,
        scratch_types=scratch,
    )
    def k(edges_hbm, dst_hbm, zeros_hbm, out_hbm, *sc):
        idx_v = sc[0:nb]
        rows_v = sc[nb:2 * nb]
        isem = sc[2 * nb:3 * nb]
        esem = sc[3 * nb:4 * nb]
        acc_sh = sc[4 * nb]
        cid = lax.axis_index("c")
        sid = lax.axis_index("s")
        wid = sid * _NUM_SC_CORES + cid
        row0 = sid * rows_per_sub
        pltpu.sync_copy(zeros_hbm.at[pl.ds(row0, rows_per_sub)],
                        acc_sh.at[pl.ds(row0, rows_per_sub)])
        plsc.subcore_barrier()
        tbase = wid * per_tile
        dbase = tbase

        for u in range(nb):
            pltpu.async_copy(dst_hbm.at[pl.ds(dbase + u * w, w)], idx_v[u],
                             isem[u])
            pltpu.async_copy(edges_hbm.at[pl.ds(tbase + u * w, w)], rows_v[u],
                             esem[u])

        @pl.loop(0, iters)
        def _(ci):
            for u in range(nb):
                off = (ci * nb + u) * w
                pltpu.make_async_copy(dst_hbm.at[pl.ds(dbase + off, w)],
                                      idx_v[u], isem[u]).wait()
                pltpu.make_async_copy(edges_hbm.at[pl.ds(tbase + off, w)],
                                      rows_v[u], esem[u]).wait()
                pltpu.sync_copy(rows_v[u], acc_sh.at[idx_v[u]], add=True)

                @pl.when(ci < iters - 1)
                def _():
                    pltpu.async_copy(
                        dst_hbm.at[pl.ds(dbase + off + nb * w, w)], idx_v[u],
                        isem[u])
                    pltpu.async_copy(
                        edges_hbm.at[pl.ds(tbase + off + nb * w, w)],
                        rows_v[u], esem[u])

        plsc.subcore_barrier()
        pltpu.sync_copy(acc_sh.at[pl.ds(row0, rows_per_sub)],
                        out_hbm.at[cid].at[pl.ds(row0, rows_per_sub)])

    return k(edges, dst, zeros)


def _node_body(*refs):
    x_ref = refs[0]
    agg_refs = refs[1:-8]
    (wa_ref, wb_ref, b1_ref, w2_ref, b2_ref, g_ref, bt_ref,
     out_ref) = refs[-8:]
    x = x_ref[...]
    agg = agg_refs[0][0]
    for r in agg_refs[1:]:
        agg = agg + r[0]
    pre = (jnp.dot(x, wa_ref[...], preferred_element_type=jnp.float32)
           + jnp.dot(agg, wb_ref[...], preferred_element_type=jnp.float32)
           + b1_ref[...])
    h = pre * jax.nn.sigmoid(pre)
    h2 = jnp.dot(h, w2_ref[...], preferred_element_type=jnp.float32) + b2_ref[...]
    mu = jnp.mean(h2, axis=-1, keepdims=True)
    zc = h2 - mu
    var = jnp.mean(zc * zc, axis=-1, keepdims=True)
    out_ref[...] = zc * lax.rsqrt(var + 1e-5) * g_ref[...] + bt_ref[...] + x


def _node_mlp(x, aggs, wa, wb, b1, w2, b2, g, bt, block=2000):
    n, c = x.shape
    row = lambda v: v.reshape(1, c)
    bspec = pl.BlockSpec((block, c), lambda i: (i, 0))
    a0 = pl.BlockSpec((1, block, c), lambda i: (0, i, 0))
    a1 = pl.BlockSpec((1, block, c), lambda i: (1, i, 0))
    wspec = pl.BlockSpec((c, c), lambda i: (0, 0))
    vspec = pl.BlockSpec((1, c), lambda i: (0, 0))
    agg_args = []
    agg_specs = []
    for a in aggs:
        agg_args += [a, a]
        agg_specs += [a0, a1]
    return pl.pallas_call(
        _node_body,
        grid=(n // block,),
        in_specs=[bspec] + agg_specs + [wspec, wspec, vspec, wspec, vspec,
                                        vspec, vspec],
        out_specs=bspec,
        out_shape=jax.ShapeDtypeStruct((n, c), jnp.float32),
    )(x, *agg_args, wa, wb, row(b1), w2, row(b2), row(g), row(bt))


def kernel(x, edge_attr, edge_index, shapes, e_W1, e_b1, e_W2, e_b2, e_g,
           e_bt, n_W1, n_b1, n_W2, n_b2, n_g, n_bt):
    n, c = x.shape
    num_layers = e_W1.shape[0]
    src = edge_index[0]
    dst = edge_index[1]
    e = src.shape[0]
    e_ch = e // _NCH
    assert e % (_NCH * 128) == 0 and e_ch % _EDGE_BLK == 0
    n_pad = ((n + 2047) // 2048) * 2048
    stride = 128 * _NUM_SC_SUBCORES * _G_NBUF
    e_pad = ((e_ch + stride - 1) // stride) * stride
    idxp = jnp.pad(jnp.stack([dst, src]).reshape(2 * _NCH, e_ch),
                   ((0, 0), (0, e_pad - e_ch)))
    zeros = jnp.zeros((n_pad, c), jnp.float32)
    x_out = x
    ea = edge_attr
    for l in range(num_layers):
        w1 = e_W1[l]
        x_pad = jnp.pad(x_out, ((0, n_pad - n), (0, 0)))
        tables = _proj(x_pad, jnp.stack([w1[:c], w1[c:2 * c]]))
        buf = _alloc_edges(e, c)
        for kc in range(_NCH):
            gab = _sc_gather(tables, idxp, kc, e_ch)
            buf = _edge_mlp(buf, gab, ea, kc * (e_ch // _EDGE_BLK),
                            w1[2 * c:], e_b1[l], e_W2[l], e_b2[l], e_g[l],
                            e_bt[l])
        agg = _sc_scatter(buf, dst, zeros)
        nw1 = n_W1[l]
        x_out = _node_mlp(x_out, [agg], nw1[:c], nw1[c:], n_b1[l], n_W2[l],
                          n_b2[l], n_g[l], n_bt[l])
        ea = buf
    return (x_out, ea)


# speedup vs baseline: 1.1131x; 1.0410x over previous
import functools

import jax
import jax.numpy as jnp
from jax import lax
from jax.experimental import pallas as pl
from jax.experimental.pallas import tpu as pltpu
from jax.experimental.pallas import tpu_sc as plsc

_NUM_SC_CORES = 2
_NUM_SC_SUBCORES = 16
_NCH = 1
_G_NBUF = 2
_S_NBUF = 2
_SCATTER_W = 40
_EDGE_BLK = 4000


def _proj_body(x_ref, w_ref, out_ref):
    out_ref[0] = jnp.dot(x_ref[...], w_ref[0],
                         preferred_element_type=jnp.float32)


def _proj(x_pad, wstack, block=2048):
    n_pad, c = x_pad.shape
    return pl.pallas_call(
        _proj_body,
        grid=(2, n_pad // block),
        in_specs=[
            pl.BlockSpec((block, c), lambda i, j: (j, 0)),
            pl.BlockSpec((1, c, c), lambda i, j: (i, 0, 0)),
        ],
        out_specs=pl.BlockSpec((1, block, c), lambda i, j: (i, j, 0)),
        out_shape=jax.ShapeDtypeStruct((2, n_pad, c), jnp.float32),
    )(x_pad, wstack)


def _sc_gather(tables, idxp, kc, e_ch):
    _, n_pad, c = tables.shape
    e_pad = idxp.shape[1]
    w = 128
    ns = _NUM_SC_SUBCORES
    nb = _G_NBUF
    per_sub = e_pad // (ns * w)
    iters = per_sub // nb
    rows_tab = n_pad // ns
    mesh = plsc.VectorSubcoreMesh(core_axis_name="c", subcore_axis_name="s")

    scratch = ([pltpu.VMEM((w,), jnp.int32)] * nb
               + [pltpu.VMEM((w, c), jnp.float32)] * nb
               + [pltpu.SemaphoreType.DMA] * (2 * nb)
               + [pltpu.VMEM_SHARED((n_pad, c), jnp.float32)])

    @functools.partial(
        pl.kernel,
        out_type=jax.ShapeDtypeStruct((_NUM_SC_CORES, e_ch, c), jnp.float32),
        mesh=mesh,
        scratch_types=scratch,
    )
    def k(tab_hbm, idx_hbm, out_hbm, *sc):
        idx_v = sc[0:nb]
        rows_v = sc[nb:2 * nb]
        isem = sc[2 * nb:3 * nb]
        osem = sc[3 * nb:4 * nb]
        tab_sh = sc[4 * nb]
        cid = lax.axis_index("c")
        sid = lax.axis_index("s")
        pltpu.sync_copy(tab_hbm.at[cid].at[pl.ds(sid * rows_tab, rows_tab)],
                        tab_sh.at[pl.ds(sid * rows_tab, rows_tab)])
        plsc.subcore_barrier()
        start = sid * per_sub

        for u in range(nb):
            pltpu.async_copy(
                idx_hbm.at[cid * _NCH + kc].at[pl.ds((start + u) * w, w)], idx_v[u],
                isem[u])

        @pl.loop(0, iters)
        def _(ci):
            for u in range(nb):
                base = (start + ci * nb + u) * w

                @pl.when(jnp.logical_and(ci > 0, base - nb * w < e_ch))
                def _():
                    pltpu.make_async_copy(
                        rows_v[u],
                        out_hbm.at[cid].at[pl.ds(base - nb * w, w)],
                        osem[u]).wait()

                pltpu.make_async_copy(
                    idx_hbm.at[cid * _NCH + kc].at[pl.ds(base, w)], idx_v[u],
                    isem[u]).wait()
                pltpu.sync_copy(tab_sh.at[idx_v[u]], rows_v[u])

                @pl.when(base < e_ch)
                def _():
                    pltpu.async_copy(
                        rows_v[u], out_hbm.at[cid].at[pl.ds(base, w)],
                        osem[u])

                @pl.when(ci < iters - 1)
                def _():
                    pltpu.async_copy(
                        idx_hbm.at[cid * _NCH + kc].at[pl.ds(base + nb * w, w)],
                        idx_v[u], isem[u])

        for u in range(nb):
            last = (start + (iters - 1) * nb + u) * w

            @pl.when(last < e_ch)
            def _():
                pltpu.make_async_copy(
                    rows_v[u], out_hbm.at[cid].at[pl.ds(last, w)],
                    osem[u]).wait()

    return k(tables, idxp)


def _alloc_body(out_ref):
    out_ref[...] = jnp.zeros_like(out_ref)


def _alloc_edges(e, c):
    return pl.pallas_call(
        _alloc_body,
        grid=(1,),
        out_specs=pl.BlockSpec((8, c), lambda i: (0, 0)),
        out_shape=jax.ShapeDtypeStruct((e, c), jnp.float32),
    )()


def _edge_body(seed_ref, ga_ref, gb_ref, ea_ref, w1_ref, b1_ref, w2_ref,
               b2_ref, g_ref, bt_ref, out_ref):
    del seed_ref
    ea = ea_ref[...]
    pre = (ga_ref[0] + gb_ref[0]
           + jnp.dot(ea, w1_ref[...], preferred_element_type=jnp.float32)
           + b1_ref[...])
    h = pre * jax.nn.sigmoid(pre)
    h2 = jnp.dot(h, w2_ref[...], preferred_element_type=jnp.float32) + b2_ref[...]
    mu = jnp.mean(h2, axis=-1, keepdims=True)
    zc = h2 - mu
    var = jnp.mean(zc * zc, axis=-1, keepdims=True)
    out_ref[...] = zc * lax.rsqrt(var + 1e-5) * g_ref[...] + bt_ref[...] + ea


def _edge_mlp(out_buf, gab, ea_arr, off_blocks, w1c, b1, w2, b2, g, bt):
    block = _EDGE_BLK
    _, e_ch, c = gab.shape
    e = ea_arr.shape[0]
    row = lambda v: v.reshape(1, c)
    seedspec = pl.BlockSpec((8, c), lambda i: (0, 0))
    aspec = pl.BlockSpec((1, block, c), lambda i: (0, i, 0))
    bspec2 = pl.BlockSpec((1, block, c), lambda i: (1, i, 0))
    easpec = pl.BlockSpec((block, c), lambda i: (i + off_blocks, 0))
    ospec = pl.BlockSpec((block, c), lambda i: (i + off_blocks, 0))
    wspec = pl.BlockSpec((c, c), lambda i: (0, 0))
    vspec = pl.BlockSpec((1, c), lambda i: (0, 0))
    return pl.pallas_call(
        _edge_body,
        grid=(e_ch // block,),
        in_specs=[seedspec, aspec, bspec2, easpec, wspec, vspec, wspec,
                  vspec, vspec, vspec],
        out_specs=ospec,
        out_shape=jax.ShapeDtypeStruct((e, c), jnp.float32),
        input_output_aliases={0: 0},
    )(out_buf, gab, gab, ea_arr, w1c, row(b1), w2, row(b2), row(g), row(bt))


def _sc_scatter(edges, dst, zeros):
    e, c = edges.shape
    n = zeros.shape[0]
    w = _SCATTER_W
    nb = _S_NBUF
    nw = _NUM_SC_CORES * _NUM_SC_SUBCORES
    per_tile = e // nw
    chunks = per_tile // w
    iters = chunks // nb
    assert chunks % nb == 0 and per_tile % w == 0
    rows_per_sub = n // _NUM_SC_SUBCORES
    assert rows_per_sub % 8 == 0
    mesh = plsc.VectorSubcoreMesh(core_axis_name="c", subcore_axis_name="s")

    scratch = ([pltpu.VMEM((w,), jnp.int32)] * nb
               + [pltpu.VMEM((w, c), jnp.float32)] * nb
               + [pltpu.SemaphoreType.DMA] * (2 * nb)
               + [pltpu.VMEM_SHARED((n, c), jnp.float32)])

    @functools.partial(
        pl.kernel,
        out_type=jax.ShapeDtypeStruct((_NUM_SC_CORES, n, c), jnp.float32),
        mesh=mesh,
        scratch_types=scratch,
    )
    def k(edges_hbm, dst_hbm, zeros_hbm, out_hbm, *sc):
        idx_v = sc[0:nb]
        rows_v = sc[nb:2 * nb]
        isem = sc[2 * nb:3 * nb]
        esem = sc[3 * nb:4 * nb]
        acc_sh = sc[4 * nb]
        cid = lax.axis_index("c")
        sid = lax.axis_index("s")
        wid = sid * _NUM_SC_CORES + cid
        row0 = sid * rows_per_sub
        pltpu.sync_copy(zeros_hbm.at[pl.ds(row0, rows_per_sub)],
                        acc_sh.at[pl.ds(row0, rows_per_sub)])
        plsc.subcore_barrier()
        tbase = wid * per_tile
        dbase = tbase

        for u in range(nb):
            pltpu.async_copy(dst_hbm.at[pl.ds(dbase + u * w, w)], idx_v[u],
                             isem[u])
            pltpu.async_copy(edges_hbm.at[pl.ds(tbase + u * w, w)], rows_v[u],
                             esem[u])

        @pl.loop(0, iters)
        def _(ci):
            for u in range(nb):
                off = (ci * nb + u) * w
                pltpu.make_async_copy(dst_hbm.at[pl.ds(dbase + off, w)],
                                      idx_v[u], isem[u]).wait()
                pltpu.make_async_copy(edges_hbm.at[pl.ds(tbase + off, w)],
                                      rows_v[u], esem[u]).wait()
                pltpu.sync_copy(rows_v[u], acc_sh.at[idx_v[u]], add=True)

                @pl.when(ci < iters - 1)
                def _():
                    pltpu.async_copy(
                        dst_hbm.at[pl.ds(dbase + off + nb * w, w)], idx_v[u],
                        isem[u])
                    pltpu.async_copy(
                        edges_hbm.at[pl.ds(tbase + off + nb * w, w)],
                        rows_v[u], esem[u])

        plsc.subcore_barrier()
        pltpu.sync_copy(acc_sh.at[pl.ds(row0, rows_per_sub)],
                        out_hbm.at[cid].at[pl.ds(row0, rows_per_sub)])

    return k(edges, dst, zeros)


def _node_body(*refs):
    x_ref = refs[0]
    agg_refs = refs[1:-8]
    (wa_ref, wb_ref, b1_ref, w2_ref, b2_ref, g_ref, bt_ref,
     out_ref) = refs[-8:]
    x = x_ref[...]
    agg = agg_refs[0][0]
    for r in agg_refs[1:]:
        agg = agg + r[0]
    pre = (jnp.dot(x, wa_ref[...], preferred_element_type=jnp.float32)
           + jnp.dot(agg, wb_ref[...], preferred_element_type=jnp.float32)
           + b1_ref[...])
    h = pre * jax.nn.sigmoid(pre)
    h2 = jnp.dot(h, w2_ref[...], preferred_element_type=jnp.float32) + b2_ref[...]
    mu = jnp.mean(h2, axis=-1, keepdims=True)
    zc = h2 - mu
    var = jnp.mean(zc * zc, axis=-1, keepdims=True)
    out_ref[...] = zc * lax.rsqrt(var + 1e-5) * g_ref[...] + bt_ref[...] + x


def _node_mlp(x, aggs, wa, wb, b1, w2, b2, g, bt, block=2000):
    n, c = x.shape
    row = lambda v: v.reshape(1, c)
    bspec = pl.BlockSpec((block, c), lambda i: (i, 0))
    a0 = pl.BlockSpec((1, block, c), lambda i: (0, i, 0))
    a1 = pl.BlockSpec((1, block, c), lambda i: (1, i, 0))
    wspec = pl.BlockSpec((c, c), lambda i: (0, 0))
    vspec = pl.BlockSpec((1, c), lambda i: (0, 0))
    agg_args = []
    agg_specs = []
    for a in aggs:
        agg_args += [a, a]
        agg_specs += [a0, a1]
    return pl.pallas_call(
        _node_body,
        grid=(n // block,),
        in_specs=[bspec] + agg_specs + [wspec, wspec, vspec, wspec, vspec,
                                        vspec, vspec],
        out_specs=bspec,
        out_shape=jax.ShapeDtypeStruct((n, c), jnp.float32),
    )(x, *agg_args, wa, wb, row(b1), w2, row(b2), row(g), row(bt))


def kernel(x, edge_attr, edge_index, shapes, e_W1, e_b1, e_W2, e_b2, e_g,
           e_bt, n_W1, n_b1, n_W2, n_b2, n_g, n_bt):
    n, c = x.shape
    num_layers = e_W1.shape[0]
    src = edge_index[0]
    dst = edge_index[1]
    e = src.shape[0]
    e_ch = e // _NCH
    assert e % (_NCH * 128) == 0 and e_ch % _EDGE_BLK == 0
    n_pad = ((n + 2047) // 2048) * 2048
    stride = 128 * _NUM_SC_SUBCORES * _G_NBUF
    e_pad = ((e_ch + stride - 1) // stride) * stride
    idxp = jnp.pad(jnp.stack([dst, src]).reshape(2 * _NCH, e_ch),
                   ((0, 0), (0, e_pad - e_ch)))
    zeros = jnp.zeros((n_pad, c), jnp.float32)
    x_out = x
    ea = edge_attr
    for l in range(num_layers):
        w1 = e_W1[l]
        x_pad = jnp.pad(x_out, ((0, n_pad - n), (0, 0)))
        tables = _proj(x_pad, jnp.stack([w1[:c], w1[c:2 * c]]))
        buf = _alloc_edges(e, c)
        for kc in range(_NCH):
            gab = _sc_gather(tables, idxp, kc, e_ch)
            buf = _edge_mlp(buf, gab, ea, kc * (e_ch // _EDGE_BLK),
                            w1[2 * c:], e_b1[l], e_W2[l], e_b2[l], e_g[l],
                            e_bt[l])
        agg = _sc_scatter(buf, dst, zeros)
        nw1 = n_W1[l]
        x_out = _node_mlp(x_out, [agg], nw1[:c], nw1[c:], n_b1[l], n_W2[l],
                          n_b2[l], n_g[l], n_bt[l])
        ea = buf
    return (x_out, ea)


# speedup vs baseline: 1.1313x; 1.0164x over previous
import functools

import jax
import jax.numpy as jnp
from jax import lax
from jax.experimental import pallas as pl
from jax.experimental.pallas import tpu as pltpu
from jax.experimental.pallas import tpu_sc as plsc

_NUM_SC_CORES = 2
_NUM_SC_SUBCORES = 16
_NCH = 1
_G_NBUF = 2
_S_NBUF = 5
_SCATTER_W = 40
_EDGE_BLK = 2000


def _proj_body(x_ref, w_ref, out_ref):
    out_ref[0] = jnp.dot(x_ref[...], w_ref[0],
                         preferred_element_type=jnp.float32)


def _proj(x_pad, wstack, block=2048):
    n_pad, c = x_pad.shape
    return pl.pallas_call(
        _proj_body,
        grid=(2, n_pad // block),
        in_specs=[
            pl.BlockSpec((block, c), lambda i, j: (j, 0)),
            pl.BlockSpec((1, c, c), lambda i, j: (i, 0, 0)),
        ],
        out_specs=pl.BlockSpec((1, block, c), lambda i, j: (i, j, 0)),
        out_shape=jax.ShapeDtypeStruct((2, n_pad, c), jnp.float32),
    )(x_pad, wstack)


def _sc_gather(tables, idxp, kc, e_ch):
    _, n_pad, c = tables.shape
    e_pad = idxp.shape[1]
    w = 128
    ns = _NUM_SC_SUBCORES
    nb = _G_NBUF
    per_sub = e_pad // (ns * w)
    iters = per_sub // nb
    rows_tab = n_pad // ns
    mesh = plsc.VectorSubcoreMesh(core_axis_name="c", subcore_axis_name="s")

    scratch = ([pltpu.VMEM((w,), jnp.int32)] * nb
               + [pltpu.VMEM((w, c), jnp.float32)] * nb
               + [pltpu.SemaphoreType.DMA] * (2 * nb)
               + [pltpu.VMEM_SHARED((n_pad, c), jnp.float32)])

    @functools.partial(
        pl.kernel,
        out_type=jax.ShapeDtypeStruct((_NUM_SC_CORES, e_ch, c), jnp.float32),
        mesh=mesh,
        scratch_types=scratch,
    )
    def k(tab_hbm, idx_hbm, out_hbm, *sc):
        idx_v = sc[0:nb]
        rows_v = sc[nb:2 * nb]
        isem = sc[2 * nb:3 * nb]
        osem = sc[3 * nb:4 * nb]
        tab_sh = sc[4 * nb]
        cid = lax.axis_index("c")
        sid = lax.axis_index("s")
        pltpu.sync_copy(tab_hbm.at[cid].at[pl.ds(sid * rows_tab, rows_tab)],
                        tab_sh.at[pl.ds(sid * rows_tab, rows_tab)])
        plsc.subcore_barrier()
        start = sid * per_sub

        for u in range(nb):
            pltpu.async_copy(
                idx_hbm.at[cid * _NCH + kc].at[pl.ds((start + u) * w, w)], idx_v[u],
                isem[u])

        @pl.loop(0, iters)
        def _(ci):
            for u in range(nb):
                base = (start + ci * nb + u) * w

                @pl.when(jnp.logical_and(ci > 0, base - nb * w < e_ch))
                def _():
                    pltpu.make_async_copy(
                        rows_v[u],
                        out_hbm.at[cid].at[pl.ds(base - nb * w, w)],
                        osem[u]).wait()

                pltpu.make_async_copy(
                    idx_hbm.at[cid * _NCH + kc].at[pl.ds(base, w)], idx_v[u],
                    isem[u]).wait()
                pltpu.sync_copy(tab_sh.at[idx_v[u]], rows_v[u])

                @pl.when(base < e_ch)
                def _():
                    pltpu.async_copy(
                        rows_v[u], out_hbm.at[cid].at[pl.ds(base, w)],
                        osem[u])

                @pl.when(ci < iters - 1)
                def _():
                    pltpu.async_copy(
                        idx_hbm.at[cid * _NCH + kc].at[pl.ds(base + nb * w, w)],
                        idx_v[u], isem[u])

        for u in range(nb):
            last = (start + (iters - 1) * nb + u) * w

            @pl.when(last < e_ch)
            def _():
                pltpu.make_async_copy(
                    rows_v[u], out_hbm.at[cid].at[pl.ds(last, w)],
                    osem[u]).wait()

    return k(tables, idxp)


def _alloc_body(out_ref):
    out_ref[...] = jnp.zeros_like(out_ref)


def _alloc_edges(e, c):
    return pl.pallas_call(
        _alloc_body,
        grid=(1,),
        out_specs=pl.BlockSpec((8, c), lambda i: (0, 0)),
        out_shape=jax.ShapeDtypeStruct((e, c), jnp.float32),
    )()


def _edge_body(seed_ref, ga_ref, gb_ref, ea_ref, w1_ref, b1_ref, w2_ref,
               b2_ref, g_ref, bt_ref, out_ref):
    del seed_ref
    ea = ea_ref[...]
    pre = (ga_ref[0] + gb_ref[0]
           + jnp.dot(ea, w1_ref[...], preferred_element_type=jnp.float32)
           + b1_ref[...])
    h = pre * jax.nn.sigmoid(pre)
    h2 = jnp.dot(h, w2_ref[...], preferred_element_type=jnp.float32) + b2_ref[...]
    mu = jnp.mean(h2, axis=-1, keepdims=True)
    zc = h2 - mu
    var = jnp.mean(zc * zc, axis=-1, keepdims=True)
    out_ref[...] = zc * lax.rsqrt(var + 1e-5) * g_ref[...] + bt_ref[...] + ea


def _edge_mlp(out_buf, gab, ea_arr, off_blocks, w1c, b1, w2, b2, g, bt):
    block = _EDGE_BLK
    _, e_ch, c = gab.shape
    e = ea_arr.shape[0]
    row = lambda v: v.reshape(1, c)
    seedspec = pl.BlockSpec((8, c), lambda i: (0, 0))
    aspec = pl.BlockSpec((1, block, c), lambda i: (0, i, 0))
    bspec2 = pl.BlockSpec((1, block, c), lambda i: (1, i, 0))
    easpec = pl.BlockSpec((block, c), lambda i: (i + off_blocks, 0))
    ospec = pl.BlockSpec((block, c), lambda i: (i + off_blocks, 0))
    wspec = pl.BlockSpec((c, c), lambda i: (0, 0))
    vspec = pl.BlockSpec((1, c), lambda i: (0, 0))
    return pl.pallas_call(
        _edge_body,
        grid=(e_ch // block,),
        in_specs=[seedspec, aspec, bspec2, easpec, wspec, vspec, wspec,
                  vspec, vspec, vspec],
        out_specs=ospec,
        out_shape=jax.ShapeDtypeStruct((e, c), jnp.float32),
        input_output_aliases={0: 0},
    )(out_buf, gab, gab, ea_arr, w1c, row(b1), w2, row(b2), row(g), row(bt))


def _sc_scatter(edges, dst, zeros):
    e, c = edges.shape
    n = zeros.shape[0]
    w = _SCATTER_W
    nb = _S_NBUF
    nw = _NUM_SC_CORES * _NUM_SC_SUBCORES
    per_tile = e // nw
    chunks = per_tile // w
    iters = chunks // nb
    assert chunks % nb == 0 and per_tile % w == 0
    rows_per_sub = n // _NUM_SC_SUBCORES
    assert rows_per_sub % 8 == 0
    mesh = plsc.VectorSubcoreMesh(core_axis_name="c", subcore_axis_name="s")

    scratch = ([pltpu.VMEM((w,), jnp.int32)] * nb
               + [pltpu.VMEM((w, c), jnp.float32)] * nb
               + [pltpu.SemaphoreType.DMA] * (2 * nb)
               + [pltpu.VMEM_SHARED((n, c), jnp.float32)])

    @functools.partial(
        pl.kernel,
        out_type=jax.ShapeDtypeStruct((_NUM_SC_CORES, n, c), jnp.float32),
        mesh=mesh,
        scratch_types=scratch,
    )
    def k(edges_hbm, dst_hbm, zeros_hbm, out_hbm, *sc):
        idx_v = sc[0:nb]
        rows_v = sc[nb:2 * nb]
        isem = sc[2 * nb:3 * nb]
        esem = sc[3 * nb:4 * nb]
        acc_sh = sc[4 * nb]
        cid = lax.axis_index("c")
        sid = lax.axis_index("s")
        wid = sid * _NUM_SC_CORES + cid
        row0 = sid * rows_per_sub
        pltpu.sync_copy(zeros_hbm.at[pl.ds(row0, rows_per_sub)],
                        acc_sh.at[pl.ds(row0, rows_per_sub)])
        plsc.subcore_barrier()
        tbase = wid * per_tile
        dbase = tbase

        for u in range(nb):
            pltpu.async_copy(dst_hbm.at[pl.ds(dbase + u * w, w)], idx_v[u],
                             isem[u])
            pltpu.async_copy(edges_hbm.at[pl.ds(tbase + u * w, w)], rows_v[u],
                             esem[u])

        @pl.loop(0, iters)
        def _(ci):
            for u in range(nb):
                off = (ci * nb + u) * w
                pltpu.make_async_copy(dst_hbm.at[pl.ds(dbase + off, w)],
                                      idx_v[u], isem[u]).wait()
                pltpu.make_async_copy(edges_hbm.at[pl.ds(tbase + off, w)],
                                      rows_v[u], esem[u]).wait()
                pltpu.sync_copy(rows_v[u], acc_sh.at[idx_v[u]], add=True)

                @pl.when(ci < iters - 1)
                def _():
                    pltpu.async_copy(
                        dst_hbm.at[pl.ds(dbase + off + nb * w, w)], idx_v[u],
                        isem[u])
                    pltpu.async_copy(
                        edges_hbm.at[pl.ds(tbase + off + nb * w, w)],
                        rows_v[u], esem[u])

        plsc.subcore_barrier()
        pltpu.sync_copy(acc_sh.at[pl.ds(row0, rows_per_sub)],
                        out_hbm.at[cid].at[pl.ds(row0, rows_per_sub)])

    return k(edges, dst, zeros)


def _node_body(*refs):
    x_ref = refs[0]
    agg_refs = refs[1:-8]
    (wa_ref, wb_ref, b1_ref, w2_ref, b2_ref, g_ref, bt_ref,
     out_ref) = refs[-8:]
    x = x_ref[...]
    agg = agg_refs[0][0]
    for r in agg_refs[1:]:
        agg = agg + r[0]
    pre = (jnp.dot(x, wa_ref[...], preferred_element_type=jnp.float32)
           + jnp.dot(agg, wb_ref[...], preferred_element_type=jnp.float32)
           + b1_ref[...])
    h = pre * jax.nn.sigmoid(pre)
    h2 = jnp.dot(h, w2_ref[...], preferred_element_type=jnp.float32) + b2_ref[...]
    mu = jnp.mean(h2, axis=-1, keepdims=True)
    zc = h2 - mu
    var = jnp.mean(zc * zc, axis=-1, keepdims=True)
    out_ref[...] = zc * lax.rsqrt(var + 1e-5) * g_ref[...] + bt_ref[...] + x


def _node_mlp(x, aggs, wa, wb, b1, w2, b2, g, bt, block=2000):
    n, c = x.shape
    row = lambda v: v.reshape(1, c)
    bspec = pl.BlockSpec((block, c), lambda i: (i, 0))
    a0 = pl.BlockSpec((1, block, c), lambda i: (0, i, 0))
    a1 = pl.BlockSpec((1, block, c), lambda i: (1, i, 0))
    wspec = pl.BlockSpec((c, c), lambda i: (0, 0))
    vspec = pl.BlockSpec((1, c), lambda i: (0, 0))
    agg_args = []
    agg_specs = []
    for a in aggs:
        agg_args += [a, a]
        agg_specs += [a0, a1]
    return pl.pallas_call(
        _node_body,
        grid=(n // block,),
        in_specs=[bspec] + agg_specs + [wspec, wspec, vspec, wspec, vspec,
                                        vspec, vspec],
        out_specs=bspec,
        out_shape=jax.ShapeDtypeStruct((n, c), jnp.float32),
    )(x, *agg_args, wa, wb, row(b1), w2, row(b2), row(g), row(bt))


def kernel(x, edge_attr, edge_index, shapes, e_W1, e_b1, e_W2, e_b2, e_g,
           e_bt, n_W1, n_b1, n_W2, n_b2, n_g, n_bt):
    n, c = x.shape
    num_layers = e_W1.shape[0]
    src = edge_index[0]
    dst = edge_index[1]
    e = src.shape[0]
    e_ch = e // _NCH
    assert e % (_NCH * 128) == 0 and e_ch % _EDGE_BLK == 0
    n_pad = ((n + 2047) // 2048) * 2048
    stride = 128 * _NUM_SC_SUBCORES * _G_NBUF
    e_pad = ((e_ch + stride - 1) // stride) * stride
    idxp = jnp.pad(jnp.stack([dst, src]).reshape(2 * _NCH, e_ch),
                   ((0, 0), (0, e_pad - e_ch)))
    zeros = jnp.zeros((n_pad, c), jnp.float32)
    x_out = x
    ea = edge_attr
    for l in range(num_layers):
        w1 = e_W1[l]
        x_pad = jnp.pad(x_out, ((0, n_pad - n), (0, 0)))
        tables = _proj(x_pad, jnp.stack([w1[:c], w1[c:2 * c]]))
        buf = _alloc_edges(e, c)
        for kc in range(_NCH):
            gab = _sc_gather(tables, idxp, kc, e_ch)
            buf = _edge_mlp(buf, gab, ea, kc * (e_ch // _EDGE_BLK),
                            w1[2 * c:], e_b1[l], e_W2[l], e_b2[l], e_g[l],
                            e_bt[l])
        agg = _sc_scatter(buf, dst, zeros)
        nw1 = n_W1[l]
        x_out = _node_mlp(x_out, [agg], nw1[:c], nw1[c:], n_b1[l], n_W2[l],
                          n_b2[l], n_g[l], n_bt[l])
        ea = buf
    return (x_out, ea)


# speedup vs baseline: 1.1332x; 1.0017x over previous
import functools

import jax
import jax.numpy as jnp
from jax import lax
from jax.experimental import pallas as pl
from jax.experimental.pallas import tpu as pltpu
from jax.experimental.pallas import tpu_sc as plsc

_NUM_SC_CORES = 2
_NUM_SC_SUBCORES = 16
_NCH = 1
_G_NBUF = 2
_S_NBUF = 5
_SCATTER_W = 40
_EDGE_BLK = 2000


def _proj_body(x_ref, w_ref, out_ref):
    out_ref[0] = jnp.dot(x_ref[...], w_ref[0],
                         preferred_element_type=jnp.float32)


def _proj(x_pad, wstack, block=2048):
    n_pad, c = x_pad.shape
    return pl.pallas_call(
        _proj_body,
        grid=(2, n_pad // block),
        in_specs=[
            pl.BlockSpec((block, c), lambda i, j: (j, 0)),
            pl.BlockSpec((1, c, c), lambda i, j: (i, 0, 0)),
        ],
        out_specs=pl.BlockSpec((1, block, c), lambda i, j: (i, j, 0)),
        out_shape=jax.ShapeDtypeStruct((2, n_pad, c), jnp.float32),
    )(x_pad, wstack)


def _sc_gather(tables, idxp, kc, e_ch):
    _, n_pad, c = tables.shape
    e_pad = idxp.shape[1]
    w = 128
    ns = _NUM_SC_SUBCORES
    nb = _G_NBUF
    per_sub = e_pad // (ns * w)
    iters = per_sub // nb
    rows_tab = n_pad // ns
    mesh = plsc.VectorSubcoreMesh(core_axis_name="c", subcore_axis_name="s")

    scratch = ([pltpu.VMEM((w,), jnp.int32)] * nb
               + [pltpu.VMEM((w, c), jnp.float32)] * nb
               + [pltpu.SemaphoreType.DMA] * (2 * nb)
               + [pltpu.VMEM_SHARED((n_pad, c), jnp.float32)])

    @functools.partial(
        pl.kernel,
        out_type=jax.ShapeDtypeStruct((_NUM_SC_CORES, e_ch, c), jnp.float32),
        mesh=mesh,
        scratch_types=scratch,
    )
    def k(tab_hbm, idx_hbm, out_hbm, *sc):
        idx_v = sc[0:nb]
        rows_v = sc[nb:2 * nb]
        isem = sc[2 * nb:3 * nb]
        osem = sc[3 * nb:4 * nb]
        tab_sh = sc[4 * nb]
        cid = lax.axis_index("c")
        sid = lax.axis_index("s")
        pltpu.sync_copy(tab_hbm.at[cid].at[pl.ds(sid * rows_tab, rows_tab)],
                        tab_sh.at[pl.ds(sid * rows_tab, rows_tab)])
        plsc.subcore_barrier()
        start = sid * per_sub

        for u in range(nb):
            pltpu.async_copy(
                idx_hbm.at[cid * _NCH + kc].at[pl.ds((start + u) * w, w)], idx_v[u],
                isem[u])

        @pl.loop(0, iters)
        def _(ci):
            for u in range(nb):
                base = (start + ci * nb + u) * w

                @pl.when(jnp.logical_and(ci > 0, base - nb * w < e_ch))
                def _():
                    pltpu.make_async_copy(
                        rows_v[u],
                        out_hbm.at[cid].at[pl.ds(base - nb * w, w)],
                        osem[u]).wait()

                pltpu.make_async_copy(
                    idx_hbm.at[cid * _NCH + kc].at[pl.ds(base, w)], idx_v[u],
                    isem[u]).wait()
                pltpu.sync_copy(tab_sh.at[idx_v[u]], rows_v[u])

                @pl.when(base < e_ch)
                def _():
                    pltpu.async_copy(
                        rows_v[u], out_hbm.at[cid].at[pl.ds(base, w)],
                        osem[u])

                @pl.when(ci < iters - 1)
                def _():
                    pltpu.async_copy(
                        idx_hbm.at[cid * _NCH + kc].at[pl.ds(base + nb * w, w)],
                        idx_v[u], isem[u])

        for u in range(nb):
            last = (start + (iters - 1) * nb + u) * w

            @pl.when(last < e_ch)
            def _():
                pltpu.make_async_copy(
                    rows_v[u], out_hbm.at[cid].at[pl.ds(last, w)],
                    osem[u]).wait()

    return k(tables, idxp)


def _edge_body(ga_ref, gb_ref, ea_ref, w1_ref, b1_ref, w2_ref,
               b2_ref, g_ref, bt_ref, out_ref):
    ea = ea_ref[...]
    pre = (ga_ref[0] + gb_ref[0]
           + jnp.dot(ea, w1_ref[...], preferred_element_type=jnp.float32)
           + b1_ref[...])
    h = pre * jax.nn.sigmoid(pre)
    h2 = jnp.dot(h, w2_ref[...], preferred_element_type=jnp.float32) + b2_ref[...]
    mu = jnp.mean(h2, axis=-1, keepdims=True)
    zc = h2 - mu
    var = jnp.mean(zc * zc, axis=-1, keepdims=True)
    out_ref[...] = zc * lax.rsqrt(var + 1e-5) * g_ref[...] + bt_ref[...] + ea


def _edge_mlp(gab, ea_arr, w1c, b1, w2, b2, g, bt):
    block = _EDGE_BLK
    _, e, c = gab.shape
    row = lambda v: v.reshape(1, c)
    aspec = pl.BlockSpec((1, block, c), lambda i: (0, i, 0))
    bspec2 = pl.BlockSpec((1, block, c), lambda i: (1, i, 0))
    bspec = pl.BlockSpec((block, c), lambda i: (i, 0))
    wspec = pl.BlockSpec((c, c), lambda i: (0, 0))
    vspec = pl.BlockSpec((1, c), lambda i: (0, 0))
    return pl.pallas_call(
        _edge_body,
        grid=(e // block,),
        in_specs=[aspec, bspec2, bspec, wspec, vspec, wspec, vspec, vspec,
                  vspec],
        out_specs=bspec,
        out_shape=jax.ShapeDtypeStruct((e, c), jnp.float32),
    )(gab, gab, ea_arr, w1c, row(b1), w2, row(b2), row(g), row(bt))


def _sc_scatter(edges, dst, zeros):
    e, c = edges.shape
    n = zeros.shape[0]
    w = _SCATTER_W
    nb = _S_NBUF
    nw = _NUM_SC_CORES * _NUM_SC_SUBCORES
    per_tile = e // nw
    chunks = per_tile // w
    iters = chunks // nb
    assert chunks % nb == 0 and per_tile % w == 0
    rows_per_sub = n // _NUM_SC_SUBCORES
    assert rows_per_sub % 8 == 0
    mesh = plsc.VectorSubcoreMesh(core_axis_name="c", subcore_axis_name="s")

    scratch = ([pltpu.VMEM((w,), jnp.int32)] * nb
               + [pltpu.VMEM((w, c), jnp.float32)] * nb
               + [pltpu.SemaphoreType.DMA] * (2 * nb)
               + [pltpu.VMEM_SHARED((n, c), jnp.float32)])

    @functools.partial(
        pl.kernel,
        out_type=jax.ShapeDtypeStruct((_NUM_SC_CORES, n, c), jnp.float32),
        mesh=mesh,
        scratch_types=scratch,
    )
    def k(edges_hbm, dst_hbm, zeros_hbm, out_hbm, *sc):
        idx_v = sc[0:nb]
        rows_v = sc[nb:2 * nb]
        isem = sc[2 * nb:3 * nb]
        esem = sc[3 * nb:4 * nb]
        acc_sh = sc[4 * nb]
        cid = lax.axis_index("c")
        sid = lax.axis_index("s")
        wid = sid * _NUM_SC_CORES + cid
        row0 = sid * rows_per_sub
        pltpu.sync_copy(zeros_hbm.at[pl.ds(row0, rows_per_sub)],
                        acc_sh.at[pl.ds(row0, rows_per_sub)])
        plsc.subcore_barrier()
        tbase = wid * per_tile
        dbase = tbase

        for u in range(nb):
            pltpu.async_copy(dst_hbm.at[pl.ds(dbase + u * w, w)], idx_v[u],
                             isem[u])
            pltpu.async_copy(edges_hbm.at[pl.ds(tbase + u * w, w)], rows_v[u],
                             esem[u])

        @pl.loop(0, iters)
        def _(ci):
            for u in range(nb):
                off = (ci * nb + u) * w
                pltpu.make_async_copy(dst_hbm.at[pl.ds(dbase + off, w)],
                                      idx_v[u], isem[u]).wait()
                pltpu.make_async_copy(edges_hbm.at[pl.ds(tbase + off, w)],
                                      rows_v[u], esem[u]).wait()
                pltpu.sync_copy(rows_v[u], acc_sh.at[idx_v[u]], add=True)

                @pl.when(ci < iters - 1)
                def _():
                    pltpu.async_copy(
                        dst_hbm.at[pl.ds(dbase + off + nb * w, w)], idx_v[u],
                        isem[u])
                    pltpu.async_copy(
                        edges_hbm.at[pl.ds(tbase + off + nb * w, w)],
                        rows_v[u], esem[u])

        plsc.subcore_barrier()
        pltpu.sync_copy(acc_sh.at[pl.ds(row0, rows_per_sub)],
                        out_hbm.at[cid].at[pl.ds(row0, rows_per_sub)])

    return k(edges, dst, zeros)


def _node_body(*refs):
    x_ref = refs[0]
    agg_refs = refs[1:-8]
    (wa_ref, wb_ref, b1_ref, w2_ref, b2_ref, g_ref, bt_ref,
     out_ref) = refs[-8:]
    x = x_ref[...]
    agg = agg_refs[0][0]
    for r in agg_refs[1:]:
        agg = agg + r[0]
    pre = (jnp.dot(x, wa_ref[...], preferred_element_type=jnp.float32)
           + jnp.dot(agg, wb_ref[...], preferred_element_type=jnp.float32)
           + b1_ref[...])
    h = pre * jax.nn.sigmoid(pre)
    h2 = jnp.dot(h, w2_ref[...], preferred_element_type=jnp.float32) + b2_ref[...]
    mu = jnp.mean(h2, axis=-1, keepdims=True)
    zc = h2 - mu
    var = jnp.mean(zc * zc, axis=-1, keepdims=True)
    out_ref[...] = zc * lax.rsqrt(var + 1e-5) * g_ref[...] + bt_ref[...] + x


def _node_mlp(x, aggs, wa, wb, b1, w2, b2, g, bt, block=2000):
    n, c = x.shape
    row = lambda v: v.reshape(1, c)
    bspec = pl.BlockSpec((block, c), lambda i: (i, 0))
    a0 = pl.BlockSpec((1, block, c), lambda i: (0, i, 0))
    a1 = pl.BlockSpec((1, block, c), lambda i: (1, i, 0))
    wspec = pl.BlockSpec((c, c), lambda i: (0, 0))
    vspec = pl.BlockSpec((1, c), lambda i: (0, 0))
    agg_args = []
    agg_specs = []
    for a in aggs:
        agg_args += [a, a]
        agg_specs += [a0, a1]
    return pl.pallas_call(
        _node_body,
        grid=(n // block,),
        in_specs=[bspec] + agg_specs + [wspec, wspec, vspec, wspec, vspec,
                                        vspec, vspec],
        out_specs=bspec,
        out_shape=jax.ShapeDtypeStruct((n, c), jnp.float32),
    )(x, *agg_args, wa, wb, row(b1), w2, row(b2), row(g), row(bt))


def kernel(x, edge_attr, edge_index, shapes, e_W1, e_b1, e_W2, e_b2, e_g,
           e_bt, n_W1, n_b1, n_W2, n_b2, n_g, n_bt):
    n, c = x.shape
    num_layers = e_W1.shape[0]
    src = edge_index[0]
    dst = edge_index[1]
    e = src.shape[0]
    e_ch = e // _NCH
    assert e % (_NCH * 128) == 0 and e_ch % _EDGE_BLK == 0
    n_pad = ((n + 2047) // 2048) * 2048
    stride = 128 * _NUM_SC_SUBCORES * _G_NBUF
    e_pad = ((e_ch + stride - 1) // stride) * stride
    idxp = jnp.pad(jnp.stack([dst, src]).reshape(2 * _NCH, e_ch),
                   ((0, 0), (0, e_pad - e_ch)))
    zeros = jnp.zeros((n_pad, c), jnp.float32)
    x_out = x
    ea = edge_attr
    for l in range(num_layers):
        w1 = e_W1[l]
        x_pad = jnp.pad(x_out, ((0, n_pad - n), (0, 0)))
        tables = _proj(x_pad, jnp.stack([w1[:c], w1[c:2 * c]]))
        gab = _sc_gather(tables, idxp, 0, e)
        buf = _edge_mlp(gab, ea, w1[2 * c:], e_b1[l], e_W2[l], e_b2[l],
                        e_g[l], e_bt[l])
        agg = _sc_scatter(buf, dst, zeros)
        nw1 = n_W1[l]
        x_out = _node_mlp(x_out, [agg], nw1[:c], nw1[c:], n_b1[l], n_W2[l],
                          n_b2[l], n_g[l], n_bt[l])
        ea = buf
    return (x_out, ea)
